# baseline (speedup 1.0000x reference)
import functools

import jax
import jax.numpy as jnp
from jax import lax
from jax.experimental import pallas as pl
from jax.experimental.pallas import tpu as pltpu
from jax.experimental.pallas import tpu_sc as plsc

NC = 2
NS = 16
L = 16
NW = NC * NS

M = 256
IDX_W = 128
NSTREAM = M // IDX_W

_GATHER_DNUMS = lax.GatherDimensionNumbers(
    offset_dims=(), collapsed_slice_dims=(0,), start_index_map=(0,))


def _bcast_lane(vec, r):
    idx = jnp.full((L, 1), r, jnp.int32)
    return lax.gather(vec, idx, _GATHER_DNUMS, (1,),
                      mode=lax.GatherScatterMode.PROMISE_IN_BOUNDS)


def _make_triline(B, N, C):
    BW = B // NW
    SUB = BW // M

    mesh = plsc.VectorSubcoreMesh(core_axis_name="c", subcore_axis_name="s",
                                  num_cores=NC, num_subcores=NS)

    set_types = (
        [pltpu.VMEM((M,), jnp.float32)] * 3
        + [pltpu.VMEM((IDX_W,), jnp.int32)] * (3 * NSTREAM)
        + [pltpu.VMEM((M,), jnp.float32)] * 3
        + [pltpu.VMEM((M, 2 * C), jnp.float32)] * 3
    )
    scratch = ([pltpu.VMEM((16,), jnp.float32)]
               + set_types + set_types
               + [pltpu.VMEM((M, C), jnp.float32)]
               + [pltpu.SemaphoreType.DMA, pltpu.SemaphoreType.DMA])

    def _split_set(scr):
        coord_refs = scr[0:3]
        idx_refs = [scr[3 + g * NSTREAM: 3 + (g + 1) * NSTREAM]
                    for g in range(3)]
        w_refs = scr[3 + 3 * NSTREAM: 6 + 3 * NSTREAM]
        bufs = scr[6 + 3 * NSTREAM: 9 + 3 * NSTREAM]
        return coord_refs, idx_refs, w_refs, bufs

    SET_LEN = 9 + 3 * NSTREAM

    @functools.partial(
        pl.kernel,
        out_type=jax.ShapeDtypeStruct((B, C), jnp.float32),
        mesh=mesh,
        scratch_types=scratch,
        compiler_params=pltpu.CompilerParams(use_tc_tiling_on_sc=False),
    )
    def triline(xs, ys, zs, xp, yp, zp, grid, out, *scr):
        gw = scr[0]
        set_a = _split_set(scr[1:1 + SET_LEN])
        set_b = _split_set(scr[1 + SET_LEN:1 + 2 * SET_LEN])
        acc = scr[1 + 2 * SET_LEN]
        sem_a = scr[2 + 2 * SET_LEN]
        sem_b = scr[3 + 2 * SET_LEN]

        wid = lax.axis_index("s") * NC + lax.axis_index("c")
        base = wid * BW

        pltpu.sync_copy(grid.at[pl.ds(0, 16)], gw)
        g16 = gw[...]
        g0 = _bcast_lane(g16, 0)
        inv_dx = 1.0 / (_bcast_lane(g16, 1) - g0)

        tables = (xp, yp, zp)

        def prep(bset, sem, s):
            coord_refs, idx_refs, w_refs, bufs = bset
            off = base + s * M
            for c_ref, src in zip(coord_refs, (xs, ys, zs)):
                pltpu.sync_copy(src.at[pl.ds(off, M)], c_ref)
            for i in range(M // L):
                sl = pl.ds(i * L, L)
                j, k = divmod(i, IDX_W // L)
                jsl = pl.ds(k * L, L)
                for a in range(3):
                    v = coord_refs[a][sl]
                    pos = (v - g0) * inv_dx
                    idx0 = jnp.clip(pos.astype(jnp.int32), 0, N - 2)
                    idx_refs[a][j][jsl] = idx0
                    w_refs[a][sl] = pos - idx0.astype(jnp.float32)
            for g in range(3):
                for j in range(NSTREAM):
                    pltpu.async_copy(tables[g].at[idx_refs[g][j]],
                                     bufs[g].at[pl.ds(j * IDX_W, IDX_W)], sem)

        def finish(bset, sem, s):
            coord_refs, idx_refs, w_refs, bufs = bset
            off = base + s * M
            for g in range(3):
                for j in range(NSTREAM):
                    pltpu.make_async_copy(
                        tables[g].at[idx_refs[g][j]],
                        bufs[g].at[pl.ds(j * IDX_W, IDX_W)], sem).wait()
            bx, by, bz = bufs

            def combine(gr, carry2):
                r0 = gr * L
                wsl = pl.ds(r0, L)
                wx16 = w_refs[0][wsl]
                wy16 = w_refs[1][wsl]
                wz16 = w_refs[2][wsl]
                for r in range(L):
                    row = r0 + r
                    wxv = _bcast_lane(wx16, r)
                    wyv = _bcast_lane(wy16, r)
                    wzv = _bcast_lane(wz16, r)
                    for h in range(C // L):
                        c0 = pl.ds(h * L, L)
                        c1 = pl.ds(C + h * L, L)
                        fx0 = bx[row, c0]
                        fx1 = bx[row, c1]
                        fy0 = by[row, c0]
                        fy1 = by[row, c1]
                        fz0 = bz[row, c0]
                        fz1 = bz[row, c1]
                        acc[row, c0] = (fx0 * (1.0 - wxv) + fx1 * wxv
                                        + fy0 * (1.0 - wyv) + fy1 * wyv
                                        + fz0 * (1.0 - wzv) + fz1 * wzv)
                return carry2

            lax.fori_loop(0, M // L, combine, 0)
            pltpu.sync_copy(acc, out.at[pl.ds(off, M)])

        prep(set_a, sem_a, 0)

        def pair(p, carry):
            s0 = 2 * p
            prep(set_b, sem_b, s0 + 1)
            finish(set_a, sem_a, s0)

            @pl.when(s0 + 2 < SUB)
            def _():
                prep(set_a, sem_a, s0 + 2)

            finish(set_b, sem_b, s0 + 1)
            return carry

        lax.fori_loop(0, SUB // 2, pair, 0)

    return triline


def kernel(coords, x_line, y_line, z_line, grid):
    B = coords.shape[0]
    N, C = x_line.shape
    xs = coords[:, 0]
    ys = coords[:, 1]
    zs = coords[:, 2]
    xp = jnp.concatenate([x_line[:-1], x_line[1:]], axis=1)
    yp = jnp.concatenate([y_line[:-1], y_line[1:]], axis=1)
    zp = jnp.concatenate([z_line[:-1], z_line[1:]], axis=1)
    fn = _make_triline(B, N, C)
    return fn(xs, ys, zs, xp, yp, zp, grid)


# speedup vs baseline: 1.1503x; 1.1503x over previous
import functools

import jax
import jax.numpy as jnp
from jax import lax
from jax.experimental import pallas as pl
from jax.experimental.pallas import tpu as pltpu
from jax.experimental.pallas import tpu_sc as plsc

NC = 2
NS = 16
L = 16
NW = NC * NS

M = 256
IDX_W = 32
NSTREAM = M // IDX_W
CONST_SLOT = M


def _make_triline(B, N, C):
    BW = B // NW
    SUB = BW // M

    mesh = plsc.VectorSubcoreMesh(core_axis_name="c", subcore_axis_name="s",
                                  num_cores=NC, num_subcores=NS)

    set_types = (
        [pltpu.VMEM((M,), jnp.float32)] * 3
        + [pltpu.VMEM((M + 2 * L,), jnp.int32)] * 6
        + [pltpu.VMEM((M,), jnp.float32)] * 3
        + [pltpu.VMEM((M,), jnp.int32)] * 3
        + [pltpu.VMEM((M + 8, C), jnp.float32)] * 6
        + [pltpu.SMEM((8,), jnp.int32)]
    )
    SET_LEN = len(set_types)
    scratch = ([pltpu.VMEM((16,), jnp.float32)]
               + set_types + set_types
               + [pltpu.VMEM((M, C), jnp.float32)]
               + [pltpu.SemaphoreType.DMA, pltpu.SemaphoreType.DMA])

    def _split_set(scr):
        coords = scr[0:3]
        ilists = [(scr[3 + 2 * a], scr[4 + 2 * a]) for a in range(3)]
        weights = scr[9:12]
        slots = scr[12:15]
        bufs = [(scr[15 + 2 * a], scr[16 + 2 * a]) for a in range(3)]
        cnts = scr[21]
        return coords, ilists, weights, slots, bufs, cnts

    @functools.partial(
        pl.kernel,
        out_type=jax.ShapeDtypeStruct((B, C), jnp.float32),
        mesh=mesh,
        scratch_types=scratch,
        compiler_params=pltpu.CompilerParams(use_tc_tiling_on_sc=False, needs_layout_passes=False),
    )
    def triline(xs, ys, zs, xl, yl, zl, grid, out, *scr):
        gw = scr[0]
        set_a = _split_set(scr[1:1 + SET_LEN])
        set_b = _split_set(scr[1 + SET_LEN:1 + 2 * SET_LEN])
        acc = scr[1 + 2 * SET_LEN]
        sem_a = scr[2 + 2 * SET_LEN]
        sem_b = scr[3 + 2 * SET_LEN]

        wid = lax.axis_index("s") * NC + lax.axis_index("c")
        base = wid * BW

        pltpu.sync_copy(grid.at[pl.ds(0, 16)], gw)
        g16 = gw[...]
        zeros16 = jnp.zeros((L,), jnp.int32)
        g0v = plsc.load_gather(gw, [zeros16])
        dxv = plsc.load_gather(gw, [zeros16 + 1]) - g0v
        inv_dx = 1.0 / dxv

        tables = (xl, yl, zl)
        iota16 = lax.iota(jnp.int32, L)

        for bset in (set_a, set_b):
            _, _, _, _, bufs, _ = bset
            for a in range(3):
                b0, b1 = bufs[a]
                pltpu.sync_copy(tables[a].at[pl.ds(N - 2, 1)],
                                b0.at[pl.ds(CONST_SLOT, 1)])
                pltpu.sync_copy(tables[a].at[pl.ds(N - 1, 1)],
                                b1.at[pl.ds(CONST_SLOT, 1)])

        def prep(bset, sem, s):
            coords, ilists, weights, slots, bufs, cnts = bset
            off = base + s * M
            for c_ref, src in zip(coords, (xs, ys, zs)):
                pltpu.sync_copy(src.at[pl.ds(off, M)], c_ref)
            lim = jnp.float32(N - 1)
            for a in range(3):
                i0_ref, i1_ref = ilists[a]
                cnt = jnp.int32(0)
                for i in range(M // L):
                    sl = pl.ds(i * L, L)
                    v = coords[a][sl]
                    pos = (v - g0v) * inv_dx
                    idx0 = jnp.clip(pos.astype(jnp.int32), 0, N - 2)
                    weights[a][sl] = pos - idx0.astype(jnp.float32)
                    m = pos < lim
                    mi = m.astype(jnp.int32)
                    pre = plsc.cumsum(mi)
                    position = cnt + pre - 1
                    slots[a][sl] = jnp.where(m, position, CONST_SLOT)
                    plsc.store_scatter(i0_ref, [position], idx0, mask=m)
                    plsc.store_scatter(i1_ref, [position], idx0 + 1, mask=m)
                    cnt = cnt + jnp.sum(mi)
                plsc.store_scatter(i0_ref, [cnt + iota16],
                                   jnp.full((L,), N - 2, jnp.int32))
                plsc.store_scatter(i0_ref, [cnt + L + iota16],
                                   jnp.full((L,), N - 2, jnp.int32))
                plsc.store_scatter(i1_ref, [cnt + iota16],
                                   jnp.full((L,), N - 1, jnp.int32))
                plsc.store_scatter(i1_ref, [cnt + L + iota16],
                                   jnp.full((L,), N - 1, jnp.int32))
                cnts[a] = cnt
                b0, b1 = bufs[a]
                for k in range(NSTREAM):
                    @pl.when(cnt > k * IDX_W)
                    def _():
                        ksl = pl.ds(k * IDX_W, IDX_W)
                        pltpu.async_copy(tables[a].at[i0_ref.at[ksl]],
                                         b0.at[ksl], sem)
                        pltpu.async_copy(tables[a].at[i1_ref.at[ksl]],
                                         b1.at[ksl], sem)

        def finish(bset, sem, s):
            coords, ilists, weights, slots, bufs, cnts = bset
            off = base + s * M
            for a in range(3):
                i0_ref, i1_ref = ilists[a]
                b0, b1 = bufs[a]
                n = cnts[a]
                for k in range(NSTREAM):
                    @pl.when(n > k * IDX_W)
                    def _():
                        ksl = pl.ds(k * IDX_W, IDX_W)
                        pltpu.make_async_copy(tables[a].at[i0_ref.at[ksl]],
                                              b0.at[ksl], sem).wait()
                        pltpu.make_async_copy(tables[a].at[i1_ref.at[ksl]],
                                              b1.at[ksl], sem).wait()
            (bx0, bx1), (by0, by1), (bz0, bz1) = bufs

            def combine(g, carry2):
                sl = pl.ds(g * L, L)
                rows = g * L + iota16
                sx = slots[0][sl]
                sy = slots[1][sl]
                sz = slots[2][sl]
                wx = weights[0][sl]
                wy = weights[1][sl]
                wz = weights[2][sl]
                ux = 1.0 - wx
                uy = 1.0 - wy
                uz = 1.0 - wz
                for c in range(C):
                    cv = jnp.full((L,), c, jnp.int32)
                    fx0 = plsc.load_gather(bx0, [sx, cv])
                    fx1 = plsc.load_gather(bx1, [sx, cv])
                    fy0 = plsc.load_gather(by0, [sy, cv])
                    fy1 = plsc.load_gather(by1, [sy, cv])
                    fz0 = plsc.load_gather(bz0, [sz, cv])
                    fz1 = plsc.load_gather(bz1, [sz, cv])
                    val = (fx0 * ux + fx1 * wx
                           + fy0 * uy + fy1 * wy
                           + fz0 * uz + fz1 * wz)
                    plsc.store_scatter(acc, [rows, cv], val)
                return carry2

            lax.fori_loop(0, M // L, combine, 0)
            pltpu.sync_copy(acc, out.at[pl.ds(off, M)])

        prep(set_a, sem_a, 0)

        def pair(p, carry):
            s0 = 2 * p
            prep(set_b, sem_b, s0 + 1)
            finish(set_a, sem_a, s0)

            @pl.when(s0 + 2 < SUB)
            def _():
                prep(set_a, sem_a, s0 + 2)

            finish(set_b, sem_b, s0 + 1)
            return carry

        lax.fori_loop(0, SUB // 2, pair, 0)

    return triline


def kernel(coords, x_line, y_line, z_line, grid):
    B = coords.shape[0]
    N, C = x_line.shape
    xs = coords[:, 0]
    ys = coords[:, 1]
    zs = coords[:, 2]
    fn = _make_triline(B, N, C)
    return fn(xs, ys, zs, x_line, y_line, z_line, grid)
